# baseline (speedup 1.0000x reference)
import dataclasses
import functools

import jax
import jax.numpy as jnp
from jax import lax
from jax.experimental import pallas as pl
from jax.experimental.pallas import tpu as pltpu
from jax.experimental.pallas import tpu_sc as plsc

N = 10000
E = 320000
IN_C = 128
HID = 256
OUT_C = 128

NC = 2
NS = 16
NW = NC * NS

NPAD = 10240
RB = NPAD // 128
RPS = NPAD // NS
CHUNK = 128
CPW = 80
EPW = CPW * CHUNK
EPS = 2 * EPW
EPAD = NW * EPW
ECH = EPAD // CHUNK
DUMMY = N
NBUF = 4

_MESH = plsc.VectorSubcoreMesh(core_axis_name="c", subcore_axis_name="s")

_SC_PARAMS = pltpu.CompilerParams()
if "needs_layout_passes" in pltpu.CompilerParams.__dataclass_fields__:
    _SC_PARAMS = dataclasses.replace(_SC_PARAMS, needs_layout_passes=False)


@functools.partial(
    pl.kernel,
    out_type=jax.ShapeDtypeStruct((NW, NPAD), jnp.float32),
    mesh=_MESH,
    scratch_types=[
        pltpu.VMEM((EPAD // NW,), jnp.int32),
        pltpu.VMEM((NPAD,), jnp.float32),
    ],
    compiler_params=_SC_PARAMS,
)
def _sc_degree(dst_hbm, out_hbm, didx, hist):
    w = lax.axis_index("s") * NC + lax.axis_index("c")
    pltpu.sync_copy(dst_hbm.at[pl.ds(w * EPW, EPW)], didx)
    zero = jnp.zeros((16,), jnp.float32)

    @pl.loop(0, NPAD, step=16)
    def _(i):
        hist[pl.ds(i, 16)] = zero

    one = jnp.ones((16,), jnp.float32)

    @pl.loop(0, EPW, step=16)
    def _(j):
        idx = didx[pl.ds(j, 16)]
        plsc.addupdate_scatter(hist, [idx], one)

    pltpu.sync_copy(hist, out_hbm.at[w])


def _make_agg(feature_split: bool):
    nch = (EPS if feature_split else EPW) // CHUNK

    @functools.partial(
        pl.kernel,
        out_type=jax.ShapeDtypeStruct((NC, NPAD, 128), jnp.float32),
        mesh=_MESH,
        scratch_types=(
            [pltpu.VMEM_SHARED((NPAD, 128), jnp.float32)]
            + [pltpu.VMEM((2, CHUNK), jnp.int32) for _ in range(2)]
            + [pltpu.VMEM((CHUNK, 128), jnp.float32) for _ in range(2)]
            + [pltpu.SemaphoreType.DMA for _ in range(6)]
        ),
    )
    def agg(tab_hbm, idx_hbm, zeros_hbm, out_hbm, *scratch):
        acc = scratch[0]
        idxb = scratch[1:3]
        rows = scratch[3:5]
        si = scratch[5:7]
        sg = scratch[7:9]
        ss = scratch[9:11]
        c = lax.axis_index("c")
        s = lax.axis_index("s")
        pltpu.sync_copy(zeros_hbm.at[pl.ds(s * RPS, RPS)],
                        acc.at[pl.ds(s * RPS, RPS)])

        if feature_split:
            ch0 = s * nch
            idx_src = idx_hbm.at[c]
        else:
            ch0 = (s * NC + c) * nch
            idx_src = idx_hbm
        plsc.subcore_barrier()

        fidx = lambda i, b: pltpu.async_copy(
            idx_src.at[ch0 + i], idxb[b], si[b])
        wait_i = lambda i, b: pltpu.make_async_copy(
            idx_src.at[ch0 + i], idxb[b], si[b]).wait()
        gather = lambda i, b: pltpu.async_copy(
            tab_hbm.at[idxb[b].at[0]], rows[b], sg[b])
        wait_g = lambda i, b: pltpu.make_async_copy(
            tab_hbm.at[idxb[b].at[0]], rows[b], sg[b]).wait()
        scat = lambda i, b: pltpu.async_copy(
            rows[b], acc.at[idxb[b].at[1]], ss[b], add=True)
        wait_s = lambda i, b: pltpu.make_async_copy(
            rows[b], acc.at[idxb[b].at[1]], ss[b]).wait()

        fidx(0, 0)
        fidx(1, 1)
        wait_i(0, 0)
        gather(0, 0)

        @pl.loop(0, nch - 2, step=2)
        def _(i0):
            for b in range(2):
                i = i0 + b
                wait_g(i, b)
                scat(i, b)
                wait_i(i + 1, 1 - b)
                gather(i + 1, 1 - b)
                wait_s(i, b)
                fidx(i + 2, b)

        i = nch - 2
        wait_g(i, 0)
        scat(i, 0)
        wait_i(i + 1, 1)
        gather(i + 1, 1)
        wait_s(i, 0)
        wait_g(i + 1, 1)
        scat(i + 1, 1)
        wait_s(i + 1, 1)

        plsc.subcore_barrier()
        pltpu.sync_copy(acc.at[pl.ds(s * RPS, RPS)],
                        out_hbm.at[c, pl.ds(s * RPS, RPS)])

    return agg


_sc_agg_edge = _make_agg(feature_split=False)
_sc_agg_feat = _make_agg(feature_split=True)


def _tc_dinv(hist):
    def body(h_ref, o_ref):
        deg = jnp.sum(h_ref[...], axis=0) + 1.0
        o_ref[...] = lax.rsqrt(deg)[None, None, :]

    return pl.pallas_call(
        body,
        grid=(RB,),
        in_specs=[pl.BlockSpec((NW, 128), lambda i: (0, i))],
        out_specs=pl.BlockSpec((1, 1, 128), lambda i: (i, 0, 0)),
        out_shape=jax.ShapeDtypeStruct((RB, 1, 128), jnp.float32),
    )(hist)


def _tc_scale(x, dinv2):
    cdim = x.shape[1]

    def body(x_ref, d_ref, o_ref):
        o_ref[...] = x_ref[...] * d_ref[...]

    return pl.pallas_call(
        body,
        grid=(RB,),
        in_specs=[pl.BlockSpec((128, cdim), lambda i: (i, 0)),
                  pl.BlockSpec((128, 1), lambda i: (i, 0))],
        out_specs=pl.BlockSpec((128, cdim), lambda i: (i, 0)),
        out_shape=jax.ShapeDtypeStruct((NPAD, cdim), jnp.float32),
    )(x, dinv2)


def _tc_layer0(s0, g0, dinv2, w0t, b0, w1t):
    def body(s_ref, g_ref, d_ref, w0_ref, b0_ref, w1_ref, o_ref):
        d = d_ref[...]
        u0 = d * (s_ref[0] + s_ref[1] + g_ref[...])
        x1 = jnp.maximum(
            jnp.dot(u0, w0_ref[...], preferred_element_type=jnp.float32)
            + b0_ref[...], 0.0)
        g1 = d * jnp.dot(x1, w1_ref[...], preferred_element_type=jnp.float32)
        o_ref[0] = g1[:, :128]
        o_ref[1] = g1[:, 128:]

    return pl.pallas_call(
        body,
        grid=(RB,),
        in_specs=[pl.BlockSpec((NC, 128, 128), lambda i: (0, i, 0)),
                  pl.BlockSpec((128, IN_C), lambda i: (i, 0)),
                  pl.BlockSpec((128, 1), lambda i: (i, 0)),
                  pl.BlockSpec((IN_C, HID), lambda i: (0, 0)),
                  pl.BlockSpec((1, HID), lambda i: (0, 0)),
                  pl.BlockSpec((HID, HID), lambda i: (0, 0))],
        out_specs=pl.BlockSpec((NC, 128, 128), lambda i: (0, i, 0)),
        out_shape=jax.ShapeDtypeStruct((NC, NPAD, 128), jnp.float32),
    )(s0, g0, dinv2, w0t, b0, w1t)


def _tc_mid(s, g, dinv2, b, wt, split_out: bool):
    kout = wt.shape[1]

    def body(s_ref, g_ref, d_ref, b_ref, w_ref, o_ref):
        d = d_ref[...]
        ua = d * (s_ref[0] + g_ref[0])
        ub = d * (s_ref[1] + g_ref[1])
        x = jnp.maximum(jnp.concatenate([ua, ub], axis=1) + b_ref[...], 0.0)
        gn = d * jnp.dot(x, w_ref[...], preferred_element_type=jnp.float32)
        if split_out:
            o_ref[0] = gn[:, :128]
            o_ref[1] = gn[:, 128:]
        else:
            o_ref[...] = gn

    if split_out:
        out_spec = pl.BlockSpec((NC, 128, 128), lambda i: (0, i, 0))
        out_shape = jax.ShapeDtypeStruct((NC, NPAD, 128), jnp.float32)
    else:
        out_spec = pl.BlockSpec((128, kout), lambda i: (i, 0))
        out_shape = jax.ShapeDtypeStruct((NPAD, kout), jnp.float32)

    return pl.pallas_call(
        body,
        grid=(RB,),
        in_specs=[pl.BlockSpec((NC, 128, 128), lambda i: (0, i, 0)),
                  pl.BlockSpec((NC, 128, 128), lambda i: (0, i, 0)),
                  pl.BlockSpec((128, 1), lambda i: (i, 0)),
                  pl.BlockSpec((1, HID), lambda i: (0, 0)),
                  pl.BlockSpec((HID, kout), lambda i: (0, 0))],
        out_specs=out_spec,
        out_shape=out_shape,
    )(s, g, dinv2, b, wt)


def _tc_final(s3, g3, dinv2, b3):
    def body(s_ref, g_ref, d_ref, b_ref, o_ref):
        o_ref[...] = (d_ref[...] * (s_ref[0] + s_ref[1] + g_ref[...])
                      + b_ref[...])

    return pl.pallas_call(
        body,
        grid=(RB,),
        in_specs=[pl.BlockSpec((NC, 128, 128), lambda i: (0, i, 0)),
                  pl.BlockSpec((128, OUT_C), lambda i: (i, 0)),
                  pl.BlockSpec((128, 1), lambda i: (i, 0)),
                  pl.BlockSpec((1, OUT_C), lambda i: (0, 0))],
        out_specs=pl.BlockSpec((128, OUT_C), lambda i: (i, 0)),
        out_shape=jax.ShapeDtypeStruct((NPAD, OUT_C), jnp.float32),
    )(s3, g3, dinv2, b3)


def kernel(x, edge_index, W0, b0, W1, b1, W2, b2, W3, b3):
    pad = jnp.full((EPAD - E,), DUMMY, dtype=jnp.int32)
    src = jnp.concatenate([edge_index[0], pad])
    dst = jnp.concatenate([edge_index[1], pad])
    src2 = src.reshape(ECH, CHUNK)
    dst2 = dst.reshape(ECH, CHUNK)
    idxe = jnp.stack([src2, dst2], axis=1)
    idxf = jnp.stack([idxe, jnp.stack([src2 + NPAD, dst2], axis=1)])
    x_pad = jnp.pad(x, ((0, NPAD - N), (0, 0)))
    zeros = jnp.zeros((NPAD, 128), jnp.float32)

    hist = _sc_degree(dst)
    dinv2 = _tc_dinv(hist).reshape(NPAD, 1)

    g0 = _tc_scale(x_pad, dinv2)
    s0 = _sc_agg_edge(g0, idxe, zeros)
    g1 = _tc_layer0(s0, g0, dinv2, W0.T, b0.reshape(1, HID), W1.T)
    s1 = _sc_agg_feat(g1.reshape(2 * NPAD, 128), idxf, zeros)
    g2 = _tc_mid(s1, g1, dinv2, b1.reshape(1, HID), W2.T, split_out=True)
    s2 = _sc_agg_feat(g2.reshape(2 * NPAD, 128), idxf, zeros)
    g3 = _tc_mid(s2, g2, dinv2, b2.reshape(1, HID), W3.T, split_out=False)
    s3 = _sc_agg_edge(g3, idxe, zeros)
    out = _tc_final(s3, g3, dinv2, b3.reshape(1, OUT_C))
    return out[:N]


# speedup vs baseline: 2.2915x; 2.2915x over previous
import dataclasses
import functools

import jax
import jax.numpy as jnp
from jax import lax
from jax.experimental import pallas as pl
from jax.experimental.pallas import tpu as pltpu
from jax.experimental.pallas import tpu_sc as plsc

N = 10000
E = 320000
IN_C = 128
HID = 256
OUT_C = 128

NC = 2
NS = 16
NW = NC * NS

NPAD = 10240
RB = NPAD // 128
RPS = NPAD // NS
CHUNK = 128
CPW = 80
EPW = CPW * CHUNK
EPS = 2 * EPW
EPAD = NW * EPW
ECH = EPAD // CHUNK
DUMMY = N
NBUF = 4

_MESH = plsc.VectorSubcoreMesh(core_axis_name="c", subcore_axis_name="s")

_SC_PARAMS = pltpu.CompilerParams()
if "needs_layout_passes" in pltpu.CompilerParams.__dataclass_fields__:
    _SC_PARAMS = dataclasses.replace(_SC_PARAMS, needs_layout_passes=False)


@functools.partial(
    pl.kernel,
    out_type=jax.ShapeDtypeStruct((NW, NPAD), jnp.float32),
    mesh=_MESH,
    scratch_types=[
        pltpu.VMEM((EPAD // NW,), jnp.int32),
        pltpu.VMEM((NPAD,), jnp.float32),
    ],
    compiler_params=_SC_PARAMS,
)
def _sc_degree(dst_hbm, out_hbm, didx, hist):
    w = lax.axis_index("s") * NC + lax.axis_index("c")
    pltpu.sync_copy(dst_hbm.at[pl.ds(w * EPW, EPW)], didx)
    zero = jnp.zeros((16,), jnp.float32)

    @pl.loop(0, NPAD, step=16)
    def _(i):
        hist[pl.ds(i, 16)] = zero

    one = jnp.ones((16,), jnp.float32)

    @pl.loop(0, EPW, step=16)
    def _(j):
        idx = didx[pl.ds(j, 16)]
        plsc.addupdate_scatter(hist, [idx], one)

    pltpu.sync_copy(hist, out_hbm.at[w])


def _make_agg(feature_split: bool):
    nch = (EPS if feature_split else EPW) // CHUNK

    @functools.partial(
        pl.kernel,
        out_type=jax.ShapeDtypeStruct((NC, NPAD, 128), jnp.float32),
        mesh=_MESH,
        scratch_types=(
            [pltpu.VMEM_SHARED((NPAD, 128), jnp.float32)]
            + [pltpu.VMEM((2, CHUNK), jnp.int32) for _ in range(2)]
            + [pltpu.VMEM((CHUNK, 128), jnp.float32) for _ in range(2)]
            + [pltpu.SemaphoreType.DMA for _ in range(6)]
        ),
    )
    def agg(tab_hbm, idx_hbm, zeros_hbm, out_hbm, *scratch):
        acc = scratch[0]
        idxb = scratch[1:3]
        rows = scratch[3:5]
        si = scratch[5:7]
        sg = scratch[7:9]
        ss = scratch[9:11]
        c = lax.axis_index("c")
        s = lax.axis_index("s")
        pltpu.sync_copy(zeros_hbm.at[pl.ds(s * RPS, RPS)],
                        acc.at[pl.ds(s * RPS, RPS)])

        if feature_split:
            ch0 = s * nch
            idx_src = idx_hbm.at[c]
        else:
            ch0 = (s * NC + c) * nch
            idx_src = idx_hbm
        plsc.subcore_barrier()

        fidx = lambda i, b: pltpu.async_copy(
            idx_src.at[ch0 + i], idxb[b], si[b])
        wait_i = lambda i, b: pltpu.make_async_copy(
            idx_src.at[ch0 + i], idxb[b], si[b]).wait()
        gather = lambda i, b: pltpu.async_copy(
            tab_hbm.at[idxb[b].at[0]], rows[b], sg[b])
        wait_g = lambda i, b: pltpu.make_async_copy(
            tab_hbm.at[idxb[b].at[0]], rows[b], sg[b]).wait()
        scat = lambda i, b: pltpu.async_copy(
            rows[b], acc.at[idxb[b].at[1]], ss[b], add=True)
        wait_s = lambda i, b: pltpu.make_async_copy(
            rows[b], acc.at[idxb[b].at[1]], ss[b]).wait()

        fidx(0, 0)
        fidx(1, 1)
        wait_i(0, 0)
        gather(0, 0)

        @pl.loop(0, nch - 2, step=2)
        def _(i0):
            for b in range(2):
                i = i0 + b
                wait_g(i, b)
                scat(i, b)
                wait_i(i + 1, 1 - b)
                gather(i + 1, 1 - b)
                wait_s(i, b)
                fidx(i + 2, b)

        i = nch - 2
        wait_g(i, 0)
        scat(i, 0)
        wait_i(i + 1, 1)
        gather(i + 1, 1)
        wait_s(i, 0)
        wait_g(i + 1, 1)
        scat(i + 1, 1)
        wait_s(i + 1, 1)

        plsc.subcore_barrier()
        pltpu.sync_copy(acc.at[pl.ds(s * RPS, RPS)],
                        out_hbm.at[c, pl.ds(s * RPS, RPS)])

    return agg


_sc_agg_edge = _make_agg(feature_split=False)
_sc_agg_feat = _make_agg(feature_split=True)


def _tc_dinv(hist):
    def body(h_ref, o_ref):
        deg = jnp.sum(h_ref[...], axis=0) + 1.0
        o_ref[...] = lax.rsqrt(deg)[None, None, :]

    return pl.pallas_call(
        body,
        grid=(RB,),
        in_specs=[pl.BlockSpec((NW, 128), lambda i: (0, i))],
        out_specs=pl.BlockSpec((1, 1, 128), lambda i: (i, 0, 0)),
        out_shape=jax.ShapeDtypeStruct((RB, 1, 128), jnp.float32),
    )(hist)


def _tc_scale(x, dinv2):
    cdim = x.shape[1]

    def body(x_ref, d_ref, o_ref):
        o_ref[...] = x_ref[...] * d_ref[...]

    return pl.pallas_call(
        body,
        grid=(RB,),
        in_specs=[pl.BlockSpec((128, cdim), lambda i: (i, 0)),
                  pl.BlockSpec((128, 1), lambda i: (i, 0))],
        out_specs=pl.BlockSpec((128, cdim), lambda i: (i, 0)),
        out_shape=jax.ShapeDtypeStruct((NPAD, cdim), jnp.float32),
    )(x, dinv2)


def _tc_layer0(s0, g0, dinv2, w0t, b0, w1t):
    def body(s_ref, g_ref, d_ref, w0_ref, b0_ref, w1_ref, o_ref):
        d = d_ref[...]
        u0 = d * (s_ref[0] + s_ref[1] + g_ref[...])
        x1 = jnp.maximum(
            jnp.dot(u0, w0_ref[...], preferred_element_type=jnp.float32)
            + b0_ref[...], 0.0)
        g1 = d * jnp.dot(x1, w1_ref[...], preferred_element_type=jnp.float32)
        o_ref[0] = g1[:, :128]
        o_ref[1] = g1[:, 128:]

    return pl.pallas_call(
        body,
        grid=(RB,),
        in_specs=[pl.BlockSpec((NC, 128, 128), lambda i: (0, i, 0)),
                  pl.BlockSpec((128, IN_C), lambda i: (i, 0)),
                  pl.BlockSpec((128, 1), lambda i: (i, 0)),
                  pl.BlockSpec((IN_C, HID), lambda i: (0, 0)),
                  pl.BlockSpec((1, HID), lambda i: (0, 0)),
                  pl.BlockSpec((HID, HID), lambda i: (0, 0))],
        out_specs=pl.BlockSpec((NC, 128, 128), lambda i: (0, i, 0)),
        out_shape=jax.ShapeDtypeStruct((NC, NPAD, 128), jnp.float32),
    )(s0, g0, dinv2, w0t, b0, w1t)


def _tc_mid(s, g, dinv2, b, wt, split_out: bool):
    kout = wt.shape[1]

    def body(s_ref, g_ref, d_ref, b_ref, w_ref, o_ref):
        d = d_ref[...]
        ua = d * (s_ref[0] + g_ref[0])
        ub = d * (s_ref[1] + g_ref[1])
        x = jnp.maximum(jnp.concatenate([ua, ub], axis=1) + b_ref[...], 0.0)
        gn = d * jnp.dot(x, w_ref[...], preferred_element_type=jnp.float32)
        if split_out:
            o_ref[0] = gn[:, :128]
            o_ref[1] = gn[:, 128:]
        else:
            o_ref[...] = gn

    if split_out:
        out_spec = pl.BlockSpec((NC, 128, 128), lambda i: (0, i, 0))
        out_shape = jax.ShapeDtypeStruct((NC, NPAD, 128), jnp.float32)
    else:
        out_spec = pl.BlockSpec((128, kout), lambda i: (i, 0))
        out_shape = jax.ShapeDtypeStruct((NPAD, kout), jnp.float32)

    return pl.pallas_call(
        body,
        grid=(RB,),
        in_specs=[pl.BlockSpec((NC, 128, 128), lambda i: (0, i, 0)),
                  pl.BlockSpec((NC, 128, 128), lambda i: (0, i, 0)),
                  pl.BlockSpec((128, 1), lambda i: (i, 0)),
                  pl.BlockSpec((1, HID), lambda i: (0, 0)),
                  pl.BlockSpec((HID, kout), lambda i: (0, 0))],
        out_specs=out_spec,
        out_shape=out_shape,
    )(s, g, dinv2, b, wt)


def _tc_final(s3, g3, dinv2, b3):
    def body(s_ref, g_ref, d_ref, b_ref, o_ref):
        o_ref[...] = (d_ref[...] * (s_ref[0] + s_ref[1] + g_ref[...])
                      + b_ref[...])

    return pl.pallas_call(
        body,
        grid=(RB,),
        in_specs=[pl.BlockSpec((NC, 128, 128), lambda i: (0, i, 0)),
                  pl.BlockSpec((128, OUT_C), lambda i: (i, 0)),
                  pl.BlockSpec((128, 1), lambda i: (i, 0)),
                  pl.BlockSpec((1, OUT_C), lambda i: (0, 0))],
        out_specs=pl.BlockSpec((128, OUT_C), lambda i: (i, 0)),
        out_shape=jax.ShapeDtypeStruct((NPAD, OUT_C), jnp.float32),
    )(s3, g3, dinv2, b3)


def kernel(x, edge_index, W0, b0, W1, b1, W2, b2, W3, b3):
    pad = DUMMY + jnp.arange(EPAD - E, dtype=jnp.int32) % (NPAD - N)
    src = jnp.concatenate([edge_index[0], pad])
    dst = jnp.concatenate([edge_index[1], pad])
    src2 = src.reshape(ECH, CHUNK)
    dst2 = dst.reshape(ECH, CHUNK)
    idxe = jnp.stack([src2, dst2], axis=1)
    idxf = jnp.stack([idxe, jnp.stack([src2 + NPAD, dst2], axis=1)])
    x_pad = jnp.pad(x, ((0, NPAD - N), (0, 0)))
    zeros = jnp.zeros((NPAD, 128), jnp.float32)

    hist = _sc_degree(dst)
    dinv2 = _tc_dinv(hist).reshape(NPAD, 1)

    g0 = _tc_scale(x_pad, dinv2)
    s0 = _sc_agg_edge(g0, idxe, zeros)
    g1 = _tc_layer0(s0, g0, dinv2, W0.T, b0.reshape(1, HID), W1.T)
    s1 = _sc_agg_feat(g1.reshape(2 * NPAD, 128), idxf, zeros)
    g2 = _tc_mid(s1, g1, dinv2, b1.reshape(1, HID), W2.T, split_out=True)
    s2 = _sc_agg_feat(g2.reshape(2 * NPAD, 128), idxf, zeros)
    g3 = _tc_mid(s2, g2, dinv2, b2.reshape(1, HID), W3.T, split_out=False)
    s3 = _sc_agg_edge(g3, idxe, zeros)
    out = _tc_final(s3, g3, dinv2, b3.reshape(1, OUT_C))
    return out[:N]


# speedup vs baseline: 2.9180x; 1.2734x over previous
import dataclasses
import functools

import jax
import jax.numpy as jnp
from jax import lax
from jax.experimental import pallas as pl
from jax.experimental.pallas import tpu as pltpu
from jax.experimental.pallas import tpu_sc as plsc

N = 10000
E = 320000
IN_C = 128
HID = 256
OUT_C = 128

NC = 2
NS = 16
NW = NC * NS

NPAD = 10240
RB = NPAD // 128
RPS = NPAD // NS
CHUNK = 128
CPW = 80
EPW = CPW * CHUNK
EPS = 2 * EPW
EPAD = NW * EPW
ECH = EPAD // CHUNK
DUMMY = N
NBUF = 4

_MESH = plsc.VectorSubcoreMesh(core_axis_name="c", subcore_axis_name="s")

_SC_PARAMS = pltpu.CompilerParams()
if "needs_layout_passes" in pltpu.CompilerParams.__dataclass_fields__:
    _SC_PARAMS = dataclasses.replace(_SC_PARAMS, needs_layout_passes=False)


@functools.partial(
    pl.kernel,
    out_type=jax.ShapeDtypeStruct((NW, NPAD), jnp.float32),
    mesh=_MESH,
    scratch_types=[
        pltpu.VMEM((EPAD // NW,), jnp.int32),
        pltpu.VMEM((NPAD,), jnp.float32),
    ],
    compiler_params=_SC_PARAMS,
)
def _sc_degree(dst_hbm, out_hbm, didx, hist):
    w = lax.axis_index("s") * NC + lax.axis_index("c")
    pltpu.sync_copy(dst_hbm.at[pl.ds(w * EPW, EPW)], didx)
    zero = jnp.zeros((16,), jnp.float32)

    @pl.loop(0, NPAD, step=16)
    def _(i):
        hist[pl.ds(i, 16)] = zero

    one = jnp.ones((16,), jnp.float32)

    @pl.loop(0, EPW, step=16)
    def _(j):
        idx = didx[pl.ds(j, 16)]
        plsc.addupdate_scatter(hist, [idx], one)

    pltpu.sync_copy(hist, out_hbm.at[w])


def _make_agg(feature_split: bool):
    nch = (EPS if feature_split else EPW) // CHUNK

    @functools.partial(
        pl.kernel,
        out_type=jax.ShapeDtypeStruct((NC, NPAD, 128), jnp.float32),
        mesh=_MESH,
        scratch_types=(
            [pltpu.VMEM_SHARED((NPAD, 128), jnp.float32)]
            + [pltpu.VMEM((2, CHUNK), jnp.int32) for _ in range(2)]
            + [pltpu.VMEM((CHUNK, 128), jnp.float32) for _ in range(2)]
            + [pltpu.SemaphoreType.DMA for _ in range(6)]
        ),
    )
    def agg(tab_hbm, idx_hbm, zeros_hbm, out_hbm, *scratch):
        acc = scratch[0]
        idxb = scratch[1:3]
        rows = scratch[3:5]
        si = scratch[5:7]
        sg = scratch[7:9]
        ss = scratch[9:11]
        c = lax.axis_index("c")
        s = lax.axis_index("s")
        pltpu.sync_copy(zeros_hbm.at[pl.ds(s * RPS, RPS)],
                        acc.at[pl.ds(s * RPS, RPS)])

        if feature_split:
            ch0 = s * nch
            idx_src = idx_hbm.at[c]
        else:
            ch0 = (s * NC + c) * nch
            idx_src = idx_hbm
        plsc.subcore_barrier()

        fidx = lambda i, b: pltpu.async_copy(
            idx_src.at[ch0 + i], idxb[b], si[b])
        wait_i = lambda i, b: pltpu.make_async_copy(
            idx_src.at[ch0 + i], idxb[b], si[b]).wait()
        gather = lambda i, b: pltpu.async_copy(
            tab_hbm.at[idxb[b].at[0]], rows[b], sg[b])
        wait_g = lambda i, b: pltpu.make_async_copy(
            tab_hbm.at[idxb[b].at[0]], rows[b], sg[b]).wait()
        scat = lambda i, b: pltpu.async_copy(
            rows[b], acc.at[idxb[b].at[1]], ss[b], add=True)
        wait_s = lambda i, b: pltpu.make_async_copy(
            rows[b], acc.at[idxb[b].at[1]], ss[b]).wait()

        fidx(0, 0)
        fidx(1, 1)
        wait_i(0, 0)
        gather(0, 0)

        @pl.loop(0, nch - 2, step=2)
        def _(i0):
            for b in range(2):
                i = i0 + b
                wait_g(i, b)
                scat(i, b)
                wait_i(i + 1, 1 - b)
                gather(i + 1, 1 - b)
                wait_s(i, b)
                fidx(i + 2, b)

        i = nch - 2
        wait_g(i, 0)
        scat(i, 0)
        wait_i(i + 1, 1)
        gather(i + 1, 1)
        wait_s(i, 0)
        wait_g(i + 1, 1)
        scat(i + 1, 1)
        wait_s(i + 1, 1)

        plsc.subcore_barrier()
        pltpu.sync_copy(acc.at[pl.ds(s * RPS, RPS)],
                        out_hbm.at[c, pl.ds(s * RPS, RPS)])

    return agg


_sc_agg_edge = _make_agg(feature_split=False)
_sc_agg_feat = _make_agg(feature_split=True)


RBLK = 2048
NRB = NPAD // RBLK

_MM = lambda a, b: lax.dot_general(
    a, b, (((1,), (1,)), ((), ())), preferred_element_type=jnp.float32)


def _tc_dinv(hist):
    def body(h_ref, o_ref):
        deg = jnp.sum(h_ref[...], axis=0) + 1.0
        o_ref[...] = lax.rsqrt(deg)[None, :]

    return pl.pallas_call(
        body,
        out_shape=jax.ShapeDtypeStruct((1, NPAD), jnp.float32),
    )(hist)


def _tc_scale(x, dinv2):
    cdim = x.shape[1]

    def body(x_ref, d_ref, o_ref):
        o_ref[...] = x_ref[...] * d_ref[...]

    return pl.pallas_call(
        body,
        grid=(NRB,),
        in_specs=[pl.BlockSpec((RBLK, cdim), lambda i: (i, 0)),
                  pl.BlockSpec((RBLK, 1), lambda i: (i, 0))],
        out_specs=pl.BlockSpec((RBLK, cdim), lambda i: (i, 0)),
        out_shape=jax.ShapeDtypeStruct((NPAD, cdim), jnp.float32),
    )(x, dinv2)


def _tc_layer0(s0, g0, dinv2, w0, b0, w1):
    def body(s_ref, g_ref, d_ref, w0_ref, b0_ref, w1_ref, o_ref):
        d = d_ref[...]
        u0 = d * (s_ref[0] + s_ref[1] + g_ref[...])
        x1 = jnp.maximum(_MM(u0, w0_ref[...]) + b0_ref[...], 0.0)
        g1 = d * _MM(x1, w1_ref[...])
        o_ref[0] = g1[:, :128]
        o_ref[1] = g1[:, 128:]

    return pl.pallas_call(
        body,
        grid=(NRB,),
        in_specs=[pl.BlockSpec((NC, RBLK, 128), lambda i: (0, i, 0)),
                  pl.BlockSpec((RBLK, IN_C), lambda i: (i, 0)),
                  pl.BlockSpec((RBLK, 1), lambda i: (i, 0)),
                  pl.BlockSpec((HID, IN_C), lambda i: (0, 0)),
                  pl.BlockSpec((1, HID), lambda i: (0, 0)),
                  pl.BlockSpec((HID, HID), lambda i: (0, 0))],
        out_specs=pl.BlockSpec((NC, RBLK, 128), lambda i: (0, i, 0)),
        out_shape=jax.ShapeDtypeStruct((NC, NPAD, 128), jnp.float32),
    )(s0, g0, dinv2, w0, b0, w1)


def _tc_mid(s, g, dinv2, b, w, split_out: bool):
    kout = w.shape[0]

    def body(s_ref, g_ref, d_ref, b_ref, w_ref, o_ref):
        d = d_ref[...]
        ua = d * (s_ref[0] + g_ref[0])
        ub = d * (s_ref[1] + g_ref[1])
        x = jnp.maximum(jnp.concatenate([ua, ub], axis=1) + b_ref[...], 0.0)
        gn = d * _MM(x, w_ref[...])
        if split_out:
            o_ref[0] = gn[:, :128]
            o_ref[1] = gn[:, 128:]
        else:
            o_ref[...] = gn

    if split_out:
        out_spec = pl.BlockSpec((NC, RBLK, 128), lambda i: (0, i, 0))
        out_shape = jax.ShapeDtypeStruct((NC, NPAD, 128), jnp.float32)
    else:
        out_spec = pl.BlockSpec((RBLK, kout), lambda i: (i, 0))
        out_shape = jax.ShapeDtypeStruct((NPAD, kout), jnp.float32)

    return pl.pallas_call(
        body,
        grid=(NRB,),
        in_specs=[pl.BlockSpec((NC, RBLK, 128), lambda i: (0, i, 0)),
                  pl.BlockSpec((NC, RBLK, 128), lambda i: (0, i, 0)),
                  pl.BlockSpec((RBLK, 1), lambda i: (i, 0)),
                  pl.BlockSpec((1, HID), lambda i: (0, 0)),
                  pl.BlockSpec((kout, HID), lambda i: (0, 0))],
        out_specs=out_spec,
        out_shape=out_shape,
    )(s, g, dinv2, b, w)


def _tc_final(s3, g3, dinv2, b3):
    def body(s_ref, g_ref, d_ref, b_ref, o_ref):
        o_ref[...] = (d_ref[...] * (s_ref[0] + s_ref[1] + g_ref[...])
                      + b_ref[...])

    return pl.pallas_call(
        body,
        grid=(NRB,),
        in_specs=[pl.BlockSpec((NC, RBLK, 128), lambda i: (0, i, 0)),
                  pl.BlockSpec((RBLK, OUT_C), lambda i: (i, 0)),
                  pl.BlockSpec((RBLK, 1), lambda i: (i, 0)),
                  pl.BlockSpec((1, OUT_C), lambda i: (0, 0))],
        out_specs=pl.BlockSpec((RBLK, OUT_C), lambda i: (i, 0)),
        out_shape=jax.ShapeDtypeStruct((NPAD, OUT_C), jnp.float32),
    )(s3, g3, dinv2, b3)


def kernel(x, edge_index, W0, b0, W1, b1, W2, b2, W3, b3):
    pad = DUMMY + jnp.arange(EPAD - E, dtype=jnp.int32) % (NPAD - N)
    src = jnp.concatenate([edge_index[0], pad])
    dst = jnp.concatenate([edge_index[1], pad])
    src2 = src.reshape(ECH, CHUNK)
    dst2 = dst.reshape(ECH, CHUNK)
    idxe = jnp.stack([src2, dst2], axis=1)
    idxf = jnp.stack([idxe, jnp.stack([src2 + NPAD, dst2], axis=1)])
    x_pad = jnp.pad(x, ((0, NPAD - N), (0, 0)))
    zeros = jnp.zeros((NPAD, 128), jnp.float32)

    hist = _sc_degree(dst)
    dinv2 = _tc_dinv(hist).reshape(NPAD, 1)

    g0 = _tc_scale(x_pad, dinv2)
    s0 = _sc_agg_edge(g0, idxe, zeros)
    g1 = _tc_layer0(s0, g0, dinv2, W0, b0.reshape(1, HID), W1)
    s1 = _sc_agg_feat(g1.reshape(2 * NPAD, 128), idxf, zeros)
    g2 = _tc_mid(s1, g1, dinv2, b1.reshape(1, HID), W2, split_out=True)
    s2 = _sc_agg_feat(g2.reshape(2 * NPAD, 128), idxf, zeros)
    g3 = _tc_mid(s2, g2, dinv2, b2.reshape(1, HID), W3, split_out=False)
    s3 = _sc_agg_edge(g3, idxe, zeros)
    out = _tc_final(s3, g3, dinv2, b3.reshape(1, OUT_C))
    return out[:N]


# speedup vs baseline: 2.9232x; 1.0018x over previous
import dataclasses
import functools

import jax
import jax.numpy as jnp
from jax import lax
from jax.experimental import pallas as pl
from jax.experimental.pallas import tpu as pltpu
from jax.experimental.pallas import tpu_sc as plsc

N = 10000
E = 320000
IN_C = 128
HID = 256
OUT_C = 128

NC = 2
NS = 16
NW = NC * NS

NPAD = 10240
RB = NPAD // 128
RPS = NPAD // NS
CHUNK = 128
CPW = 80
EPW = CPW * CHUNK
EPS = 2 * EPW
EPAD = NW * EPW
ECH = EPAD // CHUNK
DUMMY = N
NBUF = 4

_MESH = plsc.VectorSubcoreMesh(core_axis_name="c", subcore_axis_name="s")

_SC_PARAMS = pltpu.CompilerParams()
if "needs_layout_passes" in pltpu.CompilerParams.__dataclass_fields__:
    _SC_PARAMS = dataclasses.replace(_SC_PARAMS, needs_layout_passes=False)


@functools.partial(
    pl.kernel,
    out_type=jax.ShapeDtypeStruct((NW, NPAD), jnp.float32),
    mesh=_MESH,
    scratch_types=[
        pltpu.VMEM((EPAD // NW,), jnp.int32),
        pltpu.VMEM((NPAD,), jnp.float32),
    ],
    compiler_params=_SC_PARAMS,
)
def _sc_degree(dst_hbm, out_hbm, didx, hist):
    w = lax.axis_index("s") * NC + lax.axis_index("c")
    pltpu.sync_copy(dst_hbm.at[pl.ds(w * EPW, EPW)], didx)
    zero = jnp.zeros((16,), jnp.float32)

    @pl.loop(0, NPAD, step=16)
    def _(i):
        hist[pl.ds(i, 16)] = zero

    one = jnp.ones((16,), jnp.float32)

    @pl.loop(0, EPW, step=16)
    def _(j):
        idx = didx[pl.ds(j, 16)]
        plsc.addupdate_scatter(hist, [idx], one)

    pltpu.sync_copy(hist, out_hbm.at[w])


def _make_agg(feature_split: bool):
    nch = (EPS if feature_split else EPW) // CHUNK

    @functools.partial(
        pl.kernel,
        out_type=jax.ShapeDtypeStruct((NC, NPAD, 128), jnp.float32),
        mesh=_MESH,
        scratch_types=(
            [pltpu.VMEM_SHARED((NPAD, 128), jnp.float32)]
            + [pltpu.VMEM((2, CHUNK), jnp.int32) for _ in range(2)]
            + [pltpu.VMEM((CHUNK, 128), jnp.float32) for _ in range(2)]
            + [pltpu.SemaphoreType.DMA for _ in range(6)]
        ),
    )
    def agg(tab_in, idx_hbm, zeros_hbm, out_hbm, *scratch):
        acc = scratch[0]
        idxb = scratch[1:3]
        rows = scratch[3:5]
        si = scratch[5:7]
        sg = scratch[7:9]
        ss = scratch[9:11]
        c = lax.axis_index("c")
        s = lax.axis_index("s")

        if feature_split:
            ch0 = s * nch
            tab_hbm = tab_in.at[c]
        else:
            ch0 = (s * NC + c) * nch
            tab_hbm = tab_in

        fidx = lambda i, b: pltpu.async_copy(
            idx_hbm.at[ch0 + i], idxb[b], si[b])
        wait_i = lambda i, b: pltpu.make_async_copy(
            idx_hbm.at[ch0 + i], idxb[b], si[b]).wait()
        gather = lambda i, b: pltpu.async_copy(
            tab_hbm.at[idxb[b].at[0]], rows[b], sg[b])
        wait_g = lambda i, b: pltpu.make_async_copy(
            tab_hbm.at[idxb[b].at[0]], rows[b], sg[b]).wait()
        scat = lambda i, b: pltpu.async_copy(
            rows[b], acc.at[idxb[b].at[1]], ss[b], add=True)
        wait_s = lambda i, b: pltpu.make_async_copy(
            rows[b], acc.at[idxb[b].at[1]], ss[b]).wait()

        fidx(0, 0)
        fidx(1, 1)
        pltpu.sync_copy(zeros_hbm, acc.at[pl.ds(s * RPS, RPS)])
        wait_i(0, 0)
        gather(0, 0)
        plsc.subcore_barrier()

        @pl.loop(0, nch - 2, step=2)
        def _(i0):
            for b in range(2):
                i = i0 + b
                wait_g(i, b)
                scat(i, b)
                wait_i(i + 1, 1 - b)
                gather(i + 1, 1 - b)
                wait_s(i, b)
                fidx(i + 2, b)

        i = nch - 2
        wait_g(i, 0)
        scat(i, 0)
        wait_i(i + 1, 1)
        gather(i + 1, 1)
        wait_s(i, 0)
        wait_g(i + 1, 1)
        scat(i + 1, 1)
        wait_s(i + 1, 1)

        plsc.subcore_barrier()
        pltpu.sync_copy(acc.at[pl.ds(s * RPS, RPS)],
                        out_hbm.at[c, pl.ds(s * RPS, RPS)])

    return agg


_sc_agg_edge = _make_agg(feature_split=False)
_sc_agg_feat = _make_agg(feature_split=True)


RBLK = 2048
NRB = NPAD // RBLK

_MM = lambda a, b: lax.dot_general(
    a, b, (((1,), (1,)), ((), ())), preferred_element_type=jnp.float32)


def _tc_dinv(hist):
    def body(h_ref, o_ref):
        deg = jnp.sum(h_ref[...], axis=0) + 1.0
        o_ref[...] = lax.rsqrt(deg)[None, :]

    return pl.pallas_call(
        body,
        out_shape=jax.ShapeDtypeStruct((1, NPAD), jnp.float32),
    )(hist)


def _tc_scale(x, dinv2):
    cdim = x.shape[1]
    blk = N // 5

    def body(x_ref, d_ref, o_ref):
        o_ref[...] = x_ref[...] * d_ref[...]

    return pl.pallas_call(
        body,
        grid=(5,),
        in_specs=[pl.BlockSpec((blk, cdim), lambda i: (i, 0)),
                  pl.BlockSpec((blk, 1), lambda i: (i, 0))],
        out_specs=pl.BlockSpec((blk, cdim), lambda i: (i, 0)),
        out_shape=jax.ShapeDtypeStruct((NPAD, cdim), jnp.float32),
    )(x, dinv2[:N])


def _tc_layer0(s0, g0, dinv2, w0, b0, w1):
    def body(s_ref, g_ref, d_ref, w0_ref, b0_ref, w1_ref, o_ref):
        d = d_ref[...]
        u0 = d * (s_ref[0] + s_ref[1] + g_ref[...])
        x1 = jnp.maximum(_MM(u0, w0_ref[...]) + b0_ref[...], 0.0)
        g1 = d * _MM(x1, w1_ref[...])
        o_ref[0] = g1[:, :128]
        o_ref[1] = g1[:, 128:]

    return pl.pallas_call(
        body,
        grid=(NRB,),
        in_specs=[pl.BlockSpec((NC, RBLK, 128), lambda i: (0, i, 0)),
                  pl.BlockSpec((RBLK, IN_C), lambda i: (i, 0)),
                  pl.BlockSpec((RBLK, 1), lambda i: (i, 0)),
                  pl.BlockSpec((HID, IN_C), lambda i: (0, 0)),
                  pl.BlockSpec((1, HID), lambda i: (0, 0)),
                  pl.BlockSpec((HID, HID), lambda i: (0, 0))],
        out_specs=pl.BlockSpec((NC, RBLK, 128), lambda i: (0, i, 0)),
        out_shape=jax.ShapeDtypeStruct((NC, NPAD, 128), jnp.float32),
    )(s0, g0, dinv2, w0, b0, w1)


def _tc_mid(s, g, dinv2, b, w, split_out: bool):
    kout = w.shape[0]

    def body(s_ref, g_ref, d_ref, b_ref, w_ref, o_ref):
        d = d_ref[...]
        ua = d * (s_ref[0] + g_ref[0])
        ub = d * (s_ref[1] + g_ref[1])
        x = jnp.maximum(jnp.concatenate([ua, ub], axis=1) + b_ref[...], 0.0)
        gn = d * _MM(x, w_ref[...])
        if split_out:
            o_ref[0] = gn[:, :128]
            o_ref[1] = gn[:, 128:]
        else:
            o_ref[...] = gn

    if split_out:
        out_spec = pl.BlockSpec((NC, RBLK, 128), lambda i: (0, i, 0))
        out_shape = jax.ShapeDtypeStruct((NC, NPAD, 128), jnp.float32)
    else:
        out_spec = pl.BlockSpec((RBLK, kout), lambda i: (i, 0))
        out_shape = jax.ShapeDtypeStruct((NPAD, kout), jnp.float32)

    return pl.pallas_call(
        body,
        grid=(NRB,),
        in_specs=[pl.BlockSpec((NC, RBLK, 128), lambda i: (0, i, 0)),
                  pl.BlockSpec((NC, RBLK, 128), lambda i: (0, i, 0)),
                  pl.BlockSpec((RBLK, 1), lambda i: (i, 0)),
                  pl.BlockSpec((1, HID), lambda i: (0, 0)),
                  pl.BlockSpec((kout, HID), lambda i: (0, 0))],
        out_specs=out_spec,
        out_shape=out_shape,
    )(s, g, dinv2, b, w)


def _tc_final(s3, g3, dinv2, b3):
    def body(s_ref, g_ref, d_ref, b_ref, o_ref):
        o_ref[...] = (d_ref[...] * (s_ref[0] + s_ref[1] + g_ref[...])
                      + b_ref[...])

    return pl.pallas_call(
        body,
        grid=(NRB,),
        in_specs=[pl.BlockSpec((NC, RBLK, 128), lambda i: (0, i, 0)),
                  pl.BlockSpec((RBLK, OUT_C), lambda i: (i, 0)),
                  pl.BlockSpec((RBLK, 1), lambda i: (i, 0)),
                  pl.BlockSpec((1, OUT_C), lambda i: (0, 0))],
        out_specs=pl.BlockSpec((RBLK, OUT_C), lambda i: (i, 0)),
        out_shape=jax.ShapeDtypeStruct((NPAD, OUT_C), jnp.float32),
    )(s3, g3, dinv2, b3)


def kernel(x, edge_index, W0, b0, W1, b1, W2, b2, W3, b3):
    pad = DUMMY + jnp.arange(EPAD - E, dtype=jnp.int32) % (NPAD - N)
    src = jnp.concatenate([edge_index[0], pad])
    dst = jnp.concatenate([edge_index[1], pad])
    src2 = src.reshape(ECH, CHUNK)
    dst2 = dst.reshape(ECH, CHUNK)
    idxe = jnp.stack([src2, dst2], axis=1)
    zeros = jnp.zeros((RPS, 128), jnp.float32)

    hist = _sc_degree(dst)
    dinv2 = _tc_dinv(hist).reshape(NPAD, 1)

    g0 = _tc_scale(x, dinv2)
    s0 = _sc_agg_edge(g0, idxe, zeros)
    g1 = _tc_layer0(s0, g0, dinv2, W0, b0.reshape(1, HID), W1)
    s1 = _sc_agg_feat(g1, idxe, zeros)
    g2 = _tc_mid(s1, g1, dinv2, b1.reshape(1, HID), W2, split_out=True)
    s2 = _sc_agg_feat(g2, idxe, zeros)
    g3 = _tc_mid(s2, g2, dinv2, b2.reshape(1, HID), W3, split_out=False)
    s3 = _sc_agg_edge(g3, idxe, zeros)
    out = _tc_final(s3, g3, dinv2, b3.reshape(1, OUT_C))
    return out[:N]


# speedup vs baseline: 2.9482x; 1.0086x over previous
import dataclasses
import functools

import jax
import jax.numpy as jnp
from jax import lax
from jax.experimental import pallas as pl
from jax.experimental.pallas import tpu as pltpu
from jax.experimental.pallas import tpu_sc as plsc

N = 10000
E = 320000
IN_C = 128
HID = 256
OUT_C = 128

NC = 2
NS = 16
NW = NC * NS

NPAD = 10240
RB = NPAD // 128
RPS = NPAD // NS
CHUNK = 128
CPW = 80
EPW = CPW * CHUNK
EPS = 2 * EPW
EPAD = NW * EPW
ECH = EPAD // CHUNK
DUMMY = N
NBUF = 4

_MESH = plsc.VectorSubcoreMesh(core_axis_name="c", subcore_axis_name="s")

_SC_PARAMS = pltpu.CompilerParams()
if "needs_layout_passes" in pltpu.CompilerParams.__dataclass_fields__:
    _SC_PARAMS = dataclasses.replace(_SC_PARAMS, needs_layout_passes=False)


@functools.partial(
    pl.kernel,
    out_type=jax.ShapeDtypeStruct((NW, NPAD), jnp.float32),
    mesh=_MESH,
    scratch_types=[
        pltpu.VMEM((CPW, 2, CHUNK), jnp.int32),
        pltpu.VMEM((NPAD,), jnp.float32),
    ],
    compiler_params=_SC_PARAMS,
)
def _sc_degree(idx_hbm, out_hbm, didx, hist):
    w = lax.axis_index("s") * NC + lax.axis_index("c")
    pltpu.sync_copy(idx_hbm.at[pl.ds(w * CPW, CPW)], didx)
    zero = jnp.zeros((16,), jnp.float32)

    @pl.loop(0, NPAD, step=16)
    def _(i):
        hist[pl.ds(i, 16)] = zero

    one = jnp.ones((16,), jnp.float32)

    @pl.loop(0, CPW)
    def _(j):
        @pl.loop(0, CHUNK, step=16)
        def _(k):
            idx = didx[j, 1, pl.ds(k, 16)]
            plsc.addupdate_scatter(hist, [idx], one)

    pltpu.sync_copy(hist, out_hbm.at[w])


def _make_agg(feature_split: bool):
    nch = (EPS if feature_split else EPW) // CHUNK

    @functools.partial(
        pl.kernel,
        out_type=jax.ShapeDtypeStruct((NC, NPAD, 128), jnp.float32),
        mesh=_MESH,
        scratch_types=(
            [pltpu.VMEM_SHARED((NPAD, 128), jnp.float32)]
            + [pltpu.VMEM((2, CHUNK), jnp.int32) for _ in range(2)]
            + [pltpu.VMEM((CHUNK, 128), jnp.float32) for _ in range(2)]
            + [pltpu.SemaphoreType.DMA for _ in range(6)]
        ),
    )
    def agg(tab_in, idx_hbm, zeros_hbm, out_hbm, *scratch):
        acc = scratch[0]
        idxb = scratch[1:3]
        rows = scratch[3:5]
        si = scratch[5:7]
        sg = scratch[7:9]
        ss = scratch[9:11]
        c = lax.axis_index("c")
        s = lax.axis_index("s")

        if feature_split:
            ch0 = s * nch
            tab_hbm = tab_in.at[c]
        else:
            ch0 = (s * NC + c) * nch
            tab_hbm = tab_in

        fidx = lambda i, b: pltpu.async_copy(
            idx_hbm.at[ch0 + i], idxb[b], si[b])
        wait_i = lambda i, b: pltpu.make_async_copy(
            idx_hbm.at[ch0 + i], idxb[b], si[b]).wait()
        gather = lambda i, b: pltpu.async_copy(
            tab_hbm.at[idxb[b].at[0]], rows[b], sg[b])
        wait_g = lambda i, b: pltpu.make_async_copy(
            tab_hbm.at[idxb[b].at[0]], rows[b], sg[b]).wait()
        scat = lambda i, b: pltpu.async_copy(
            rows[b], acc.at[idxb[b].at[1]], ss[b], add=True)
        wait_s = lambda i, b: pltpu.make_async_copy(
            rows[b], acc.at[idxb[b].at[1]], ss[b]).wait()

        fidx(0, 0)
        fidx(1, 1)
        pltpu.sync_copy(zeros_hbm, acc.at[pl.ds(s * RPS, RPS)])
        wait_i(0, 0)
        gather(0, 0)
        plsc.subcore_barrier()

        @pl.loop(0, nch - 2, step=2)
        def _(i0):
            for b in range(2):
                i = i0 + b
                wait_g(i, b)
                scat(i, b)
                wait_i(i + 1, 1 - b)
                gather(i + 1, 1 - b)
                wait_s(i, b)
                fidx(i + 2, b)

        i = nch - 2
        wait_g(i, 0)
        scat(i, 0)
        wait_i(i + 1, 1)
        gather(i + 1, 1)
        wait_s(i, 0)
        wait_g(i + 1, 1)
        scat(i + 1, 1)
        wait_s(i + 1, 1)

        plsc.subcore_barrier()
        pltpu.sync_copy(acc.at[pl.ds(s * RPS, RPS)],
                        out_hbm.at[c, pl.ds(s * RPS, RPS)])

    return agg


_sc_agg_edge = _make_agg(feature_split=False)
_sc_agg_feat = _make_agg(feature_split=True)


RBLK = 2048
NRB = NPAD // RBLK

_MM = lambda a, b: lax.dot_general(
    a, b, (((1,), (1,)), ((), ())), preferred_element_type=jnp.float32)


def _tc_dinv(hist):
    def body(h_ref, o_ref):
        deg = jnp.sum(h_ref[...], axis=0) + 1.0
        o_ref[...] = lax.rsqrt(deg)[None, :]

    return pl.pallas_call(
        body,
        out_shape=jax.ShapeDtypeStruct((1, NPAD), jnp.float32),
    )(hist)


def _tc_scale(x, dinv2):
    cdim = x.shape[1]
    blk = N // 5

    def body(x_ref, d_ref, o_ref):
        o_ref[...] = x_ref[...] * d_ref[...]

    return pl.pallas_call(
        body,
        grid=(5,),
        in_specs=[pl.BlockSpec((blk, cdim), lambda i: (i, 0)),
                  pl.BlockSpec((blk, 1), lambda i: (i, 0))],
        out_specs=pl.BlockSpec((blk, cdim), lambda i: (i, 0)),
        out_shape=jax.ShapeDtypeStruct((NPAD, cdim), jnp.float32),
    )(x, dinv2[:N])


def _tc_layer0(s0, g0, dinv2, w0, b0, w1):
    def body(s_ref, g_ref, d_ref, w0_ref, b0_ref, w1_ref, o_ref):
        d = d_ref[...]
        u0 = d * (s_ref[0] + s_ref[1] + g_ref[...])
        x1 = jnp.maximum(_MM(u0, w0_ref[...]) + b0_ref[...], 0.0)
        g1 = d * _MM(x1, w1_ref[...])
        o_ref[0] = g1[:, :128]
        o_ref[1] = g1[:, 128:]

    return pl.pallas_call(
        body,
        grid=(NRB,),
        in_specs=[pl.BlockSpec((NC, RBLK, 128), lambda i: (0, i, 0)),
                  pl.BlockSpec((RBLK, IN_C), lambda i: (i, 0)),
                  pl.BlockSpec((RBLK, 1), lambda i: (i, 0)),
                  pl.BlockSpec((HID, IN_C), lambda i: (0, 0)),
                  pl.BlockSpec((1, HID), lambda i: (0, 0)),
                  pl.BlockSpec((HID, HID), lambda i: (0, 0))],
        out_specs=pl.BlockSpec((NC, RBLK, 128), lambda i: (0, i, 0)),
        out_shape=jax.ShapeDtypeStruct((NC, NPAD, 128), jnp.float32),
    )(s0, g0, dinv2, w0, b0, w1)


def _tc_mid(s, g, dinv2, b, w, split_out: bool):
    kout = w.shape[0]

    def body(s_ref, g_ref, d_ref, b_ref, w_ref, o_ref):
        d = d_ref[...]
        ua = d * (s_ref[0] + g_ref[0])
        ub = d * (s_ref[1] + g_ref[1])
        x = jnp.maximum(jnp.concatenate([ua, ub], axis=1) + b_ref[...], 0.0)
        gn = d * _MM(x, w_ref[...])
        if split_out:
            o_ref[0] = gn[:, :128]
            o_ref[1] = gn[:, 128:]
        else:
            o_ref[...] = gn

    if split_out:
        out_spec = pl.BlockSpec((NC, RBLK, 128), lambda i: (0, i, 0))
        out_shape = jax.ShapeDtypeStruct((NC, NPAD, 128), jnp.float32)
    else:
        out_spec = pl.BlockSpec((RBLK, kout), lambda i: (i, 0))
        out_shape = jax.ShapeDtypeStruct((NPAD, kout), jnp.float32)

    return pl.pallas_call(
        body,
        grid=(NRB,),
        in_specs=[pl.BlockSpec((NC, RBLK, 128), lambda i: (0, i, 0)),
                  pl.BlockSpec((NC, RBLK, 128), lambda i: (0, i, 0)),
                  pl.BlockSpec((RBLK, 1), lambda i: (i, 0)),
                  pl.BlockSpec((1, HID), lambda i: (0, 0)),
                  pl.BlockSpec((kout, HID), lambda i: (0, 0))],
        out_specs=out_spec,
        out_shape=out_shape,
    )(s, g, dinv2, b, w)


def _tc_final(s3, g3, dinv2, b3):
    def body(s_ref, g_ref, d_ref, b_ref, o_ref):
        o_ref[...] = (d_ref[...] * (s_ref[0] + s_ref[1] + g_ref[...])
                      + b_ref[...])

    return pl.pallas_call(
        body,
        grid=(NRB,),
        in_specs=[pl.BlockSpec((NC, RBLK, 128), lambda i: (0, i, 0)),
                  pl.BlockSpec((RBLK, OUT_C), lambda i: (i, 0)),
                  pl.BlockSpec((RBLK, 1), lambda i: (i, 0)),
                  pl.BlockSpec((1, OUT_C), lambda i: (0, 0))],
        out_specs=pl.BlockSpec((RBLK, OUT_C), lambda i: (i, 0)),
        out_shape=jax.ShapeDtypeStruct((NPAD, OUT_C), jnp.float32),
    )(s3, g3, dinv2, b3)


def kernel(x, edge_index, W0, b0, W1, b1, W2, b2, W3, b3):
    pad = (DUMMY + jnp.arange(EPAD - E, dtype=jnp.int32) % (NPAD - N))
    pad2 = jnp.broadcast_to(pad.reshape(1, -1, CHUNK),
                            (2, (EPAD - E) // CHUNK, CHUNK))
    idx_all = jnp.concatenate(
        [edge_index.reshape(2, E // CHUNK, CHUNK), pad2], axis=1)
    idxe = jnp.swapaxes(idx_all, 0, 1)
    zeros = jnp.zeros((RPS, 128), jnp.float32)

    hist = _sc_degree(idxe)
    dinv2 = _tc_dinv(hist).reshape(NPAD, 1)

    g0 = _tc_scale(x, dinv2)
    s0 = _sc_agg_edge(g0, idxe, zeros)
    g1 = _tc_layer0(s0, g0, dinv2, W0, b0.reshape(1, HID), W1)
    s1 = _sc_agg_feat(g1, idxe, zeros)
    g2 = _tc_mid(s1, g1, dinv2, b1.reshape(1, HID), W2, split_out=True)
    s2 = _sc_agg_feat(g2, idxe, zeros)
    g3 = _tc_mid(s2, g2, dinv2, b2.reshape(1, HID), W3, split_out=False)
    s3 = _sc_agg_edge(g3, idxe, zeros)
    out = _tc_final(s3, g3, dinv2, b3.reshape(1, OUT_C))
    return out[:N]
